# baseline (speedup 1.0000x reference)
import functools

import jax
import jax.numpy as jnp
from jax import lax
from jax.experimental import pallas as pl
from jax.experimental.pallas import tpu as pltpu
from jax.experimental.pallas import tpu_sc as plsc

_NUM_ROUTED = 8
_NUM_TOTAL = 10
_D_MODEL = 1024
_D_FF = 512
_N_TOKENS = 2048
_SCALE = 2.5
_LANES = 128

_TILE = 256
_GRID_TILES = 24
_MAX_PAD = _TILE * _GRID_TILES
_XG_ROWS = _MAX_PAD + 256
_RBLK = 256
_NRB = _N_TOKENS // _RBLK


def _route_body(wrt_ref, bias_ref, x_ref,
                ssc0_ref, ssc1_ref, scb0_ref, scb1_ref,
                w0_ref, w1_ref, xz_ref, te_ref, nt_ref,
                ids_s):
    neg = jnp.float32(-1e30)
    cnt = jnp.zeros((1, _LANES), jnp.float32)

    for tb in range(_NRB):
        rows = pl.ds(tb * _RBLK, _RBLK)
        xb = x_ref[rows, :]
        logits = jnp.dot(xb, wrt_ref[:], preferred_element_type=jnp.float32)
        lane = jax.lax.broadcasted_iota(jnp.int32, logits.shape, 1)
        valid = lane < _NUM_TOTAL
        lm = jnp.where(valid, logits, neg)
        m = jnp.max(lm, axis=-1, keepdims=True)
        p = jnp.where(valid, jnp.exp(lm - m), 0.0)
        scores = p / jnp.sum(p, axis=-1, keepdims=True)
        sel = jnp.where(valid, scores + bias_ref[:], neg)
        m1 = jnp.max(sel, axis=-1, keepdims=True)
        i1 = jnp.min(jnp.where(sel == m1, lane, _LANES), axis=-1, keepdims=True)
        w1v = jnp.sum(jnp.where(lane == i1, scores, 0.0), axis=-1, keepdims=True)
        sel2 = jnp.where(lane == i1, neg, sel)
        m2 = jnp.max(sel2, axis=-1, keepdims=True)
        i2 = jnp.min(jnp.where(sel2 == m2, lane, _LANES), axis=-1, keepdims=True)
        w2v = jnp.sum(jnp.where(lane == i2, scores, 0.0), axis=-1, keepdims=True)
        r1 = i1 < _NUM_ROUTED
        r2 = i2 < _NUM_ROUTED
        ones = jnp.ones((1, _LANES), jnp.float32)
        w0_ref[rows, :] = jnp.where(r1, _SCALE * w1v, 0.0) * ones
        w1_ref[rows, :] = jnp.where(r2, _SCALE * w2v, 0.0) * ones
        zw = _SCALE * (jnp.where(r1, 0.0, w1v) + jnp.where(r2, 0.0, w2v))
        xz_ref[rows, :] = zw * xb
        ids_s[rows, :] = i1
        ids_s[pl.ds(_N_TOKENS + tb * _RBLK, _RBLK), :] = i2
        cnt = cnt + jnp.sum((lane == i1).astype(jnp.float32)
                            + (lane == i2).astype(jnp.float32),
                            axis=0, keepdims=True)

    lane_r = jax.lax.broadcasted_iota(jnp.int32, (1, _LANES), 1)
    pc = jnp.where(lane_r < _NUM_ROUTED,
                   jnp.ceil(cnt / _TILE) * _TILE, 0.0)
    rowi = jax.lax.broadcasted_iota(jnp.int32, (_LANES, _LANES), 0)
    coli = jax.lax.broadcasted_iota(jnp.int32, (_LANES, _LANES), 1)
    upper = (rowi < coli).astype(jnp.float32)
    off = jnp.dot(pc, upper, preferred_element_type=jnp.float32)
    total = off[:, _NUM_ROUTED:_NUM_ROUTED + 1]
    nt_ref[:] = (total / _TILE).astype(jnp.int32)

    te = jnp.zeros((1, _LANES), jnp.float32)
    for e in range(_NUM_ROUTED):
        off_e = off[:, e:e + 1] / _TILE
        te = te + (lane_r.astype(jnp.float32) >= off_e).astype(jnp.float32)
    te_ref[:] = jnp.clip(te - 1.0, 0.0, float(_NUM_ROUTED - 1)).astype(jnp.int32)

    rowb = jax.lax.broadcasted_iota(jnp.int32, (_RBLK, _RBLK), 0)
    colb = jax.lax.broadcasted_iota(jnp.int32, (_RBLK, _RBLK), 1)
    strict_low = (colb < rowb).astype(jnp.float32)
    carry = jnp.zeros((1, _LANES), jnp.float32)
    dump = _MAX_PAD + jax.lax.broadcasted_iota(jnp.int32, (_RBLK, 1), 0)
    for b in range(2 * _NRB):
        ids = ids_s[pl.ds(b * _RBLK, _RBLK), :]
        lane = jax.lax.broadcasted_iota(jnp.int32, (_RBLK, _LANES), 1)
        oh = (lane == ids).astype(jnp.float32)
        prefix = jnp.dot(strict_low, oh, preferred_element_type=jnp.float32)
        grank = jnp.sum(jnp.where(lane == ids, prefix + carry, 0.0),
                        axis=-1, keepdims=True)
        offsel = jnp.sum(jnp.where(lane == ids, off, 0.0),
                         axis=-1, keepdims=True)
        carry = carry + jnp.sum(oh, axis=0, keepdims=True)
        slot = (offsel + grank).astype(jnp.int32)
        routed = ids < _NUM_ROUTED
        ssc = jnp.where(routed, slot, dump)
        scb = jnp.where(routed, slot, dump)
        rows = pl.ds((b % _NRB) * _RBLK, _RBLK)
        if b < _NRB:
            ssc0_ref[rows, :] = ssc
            scb0_ref[rows, :] = scb
        else:
            ssc1_ref[rows, :] = ssc
            scb1_ref[rows, :] = scb


def _route(x, wrt, bias_pad):
    i32 = jnp.int32
    f32 = jnp.float32
    outs = pl.pallas_call(
        _route_body,
        in_specs=[
            pl.BlockSpec((_D_MODEL, _LANES), lambda: (0, 0)),
            pl.BlockSpec((1, _LANES), lambda: (0, 0)),
            pl.BlockSpec((_N_TOKENS, _D_MODEL), lambda: (0, 0)),
        ],
        out_specs=[
            pl.BlockSpec((_N_TOKENS, 1), lambda: (0, 0)),
            pl.BlockSpec((_N_TOKENS, 1), lambda: (0, 0)),
            pl.BlockSpec((_N_TOKENS, 1), lambda: (0, 0)),
            pl.BlockSpec((_N_TOKENS, 1), lambda: (0, 0)),
            pl.BlockSpec((_N_TOKENS, _LANES), lambda: (0, 0)),
            pl.BlockSpec((_N_TOKENS, _LANES), lambda: (0, 0)),
            pl.BlockSpec((_N_TOKENS, _D_MODEL), lambda: (0, 0)),
            pl.BlockSpec((1, _LANES), lambda: (0, 0)),
            pl.BlockSpec((1, 1), lambda: (0, 0)),
        ],
        out_shape=[
            jax.ShapeDtypeStruct((_N_TOKENS, 1), i32),
            jax.ShapeDtypeStruct((_N_TOKENS, 1), i32),
            jax.ShapeDtypeStruct((_N_TOKENS, 1), i32),
            jax.ShapeDtypeStruct((_N_TOKENS, 1), i32),
            jax.ShapeDtypeStruct((_N_TOKENS, _LANES), f32),
            jax.ShapeDtypeStruct((_N_TOKENS, _LANES), f32),
            jax.ShapeDtypeStruct((_N_TOKENS, _D_MODEL), f32),
            jax.ShapeDtypeStruct((1, _LANES), i32),
            jax.ShapeDtypeStruct((1, 1), i32),
        ],
        scratch_shapes=[pltpu.VMEM((2 * _N_TOKENS, 1), i32)],
    )(wrt, bias_pad, x)
    return outs


_NW = 32
_TPW = _N_TOKENS // _NW
_BCH = 32


@functools.cache
def _make_dispatch():
    mesh = plsc.VectorSubcoreMesh(core_axis_name="c", subcore_axis_name="s")

    @functools.partial(
        pl.kernel,
        out_type=[
            jax.ShapeDtypeStruct((_XG_ROWS, _D_MODEL), jnp.float32),
            jax.ShapeDtypeStruct((_XG_ROWS, _LANES), jnp.float32),
        ],
        mesh=mesh,
        scratch_types=[
            pltpu.VMEM((_TPW,), jnp.int32),
            pltpu.VMEM((_TPW,), jnp.int32),
            pltpu.VMEM((_TPW, _LANES), jnp.float32),
            pltpu.VMEM((_TPW, _LANES), jnp.float32),
            pltpu.VMEM((_TPW, _D_MODEL), jnp.float32),
            pltpu.SemaphoreType.DMA,
        ],
    )
    def _dispatch(s0_hbm, s1_hbm, w0_hbm, w1_hbm, x_hbm, xg_hbm, ws_hbm,
                  idx0_v, idx1_v, w0_v, w1_v, rows_v, sem):
        wid = lax.axis_index("s") * 2 + lax.axis_index("c")
        base = wid * _TPW
        loads = [
            pltpu.async_copy(s0_hbm.at[pl.ds(base, _TPW)], idx0_v, sem),
            pltpu.async_copy(s1_hbm.at[pl.ds(base, _TPW)], idx1_v, sem),
            pltpu.async_copy(w0_hbm.at[pl.ds(base, _TPW)], w0_v, sem),
            pltpu.async_copy(w1_hbm.at[pl.ds(base, _TPW)], w1_v, sem),
            pltpu.async_copy(x_hbm.at[pl.ds(base, _TPW)], rows_v, sem),
        ]
        for cp in loads:
            cp.wait()
        stores = [
            pltpu.async_copy(rows_v, xg_hbm.at[idx0_v], sem),
            pltpu.async_copy(rows_v, xg_hbm.at[idx1_v], sem),
            pltpu.async_copy(w0_v, ws_hbm.at[idx0_v], sem),
            pltpu.async_copy(w1_v, ws_hbm.at[idx1_v], sem),
        ]
        for cp in stores:
            cp.wait()

    return _dispatch


def _ffn_body(te_ref, nt_ref, xg_ref, ws_ref, wg_ref, wu_ref, wd_ref, y_ref):
    i = pl.program_id(0)

    @pl.when(i < nt_ref[0])
    def _compute():
        xb = xg_ref[:].astype(jnp.bfloat16)
        g = jnp.dot(xb, wg_ref[0], preferred_element_type=jnp.float32)
        u = jnp.dot(xb, wu_ref[0], preferred_element_type=jnp.float32)
        h = (g * jax.nn.sigmoid(g) * u).astype(jnp.bfloat16)
        y = jnp.dot(h, wd_ref[0], preferred_element_type=jnp.float32)
        y_ref[:] = y * ws_ref[:, 0:1]

    @pl.when(i == nt_ref[0])
    def _inactive():
        y_ref[:] = jnp.zeros_like(y_ref)


def _grouped_ffn(te, nt, xg, ws, w_gate, w_up, w_down):
    def _last(i, nt_ref):
        return jnp.minimum(i, jnp.maximum(nt_ref[0] - 1, 0))

    grid_spec = pltpu.PrefetchScalarGridSpec(
        num_scalar_prefetch=2,
        grid=(_GRID_TILES + 1,),
        in_specs=[
            pl.BlockSpec((_TILE, _D_MODEL),
                         lambda i, te_r, nt_r: (_last(i, nt_r), 0)),
            pl.BlockSpec((_TILE, _LANES),
                         lambda i, te_r, nt_r: (_last(i, nt_r), 0)),
            pl.BlockSpec((1, _D_MODEL, _D_FF),
                         lambda i, te_r, nt_r: (te_r[_last(i, nt_r)], 0, 0)),
            pl.BlockSpec((1, _D_MODEL, _D_FF),
                         lambda i, te_r, nt_r: (te_r[_last(i, nt_r)], 0, 0)),
            pl.BlockSpec((1, _D_FF, _D_MODEL),
                         lambda i, te_r, nt_r: (te_r[_last(i, nt_r)], 0, 0)),
        ],
        out_specs=pl.BlockSpec(
            (_TILE, _D_MODEL),
            lambda i, te_r, nt_r: (jnp.where(i < nt_r[0], i, _GRID_TILES), 0)),
    )
    return pl.pallas_call(
        _ffn_body,
        grid_spec=grid_spec,
        out_shape=jax.ShapeDtypeStruct((_XG_ROWS, _D_MODEL), jnp.float32),
    )(te, nt, xg, ws, w_gate, w_up, w_down)


_DCH = 16


@functools.cache
def _make_combine():
    mesh = plsc.VectorSubcoreMesh(core_axis_name="c", subcore_axis_name="s")

    @functools.partial(
        pl.kernel,
        out_type=jax.ShapeDtypeStruct((_N_TOKENS, _D_MODEL), jnp.float32),
        mesh=mesh,
        scratch_types=[
            pltpu.VMEM((_TPW,), jnp.int32),
            pltpu.VMEM((_TPW,), jnp.int32),
            pltpu.VMEM((_DCH, _D_MODEL), jnp.float32),
            pltpu.VMEM((_DCH, _D_MODEL), jnp.float32),
            pltpu.VMEM((_DCH, _D_MODEL), jnp.float32),
            pltpu.VMEM((_DCH, _D_MODEL), jnp.float32),
            pltpu.VMEM((_DCH, _D_MODEL), jnp.float32),
            pltpu.VMEM((_DCH, _D_MODEL), jnp.float32),
            pltpu.SemaphoreType.DMA,
            pltpu.SemaphoreType.DMA,
            pltpu.SemaphoreType.DMA,
            pltpu.SemaphoreType.DMA,
        ],
    )
    def _combine(cb0_hbm, cb1_hbm, xz_hbm, y_hbm, out_hbm,
                 cb0_v, cb1_v, xb0_v, xb1_v, y00_v, y01_v, y10_v, y11_v,
                 sl0, sl1, ss0, ss1):
        wid = lax.axis_index("s") * 2 + lax.axis_index("c")
        base = wid * _TPW
        xb = [xb0_v, xb1_v]
        y0 = [y00_v, y01_v]
        y1 = [y10_v, y11_v]
        sl = [sl0, sl1]
        ss = [ss0, ss1]
        nch = _TPW // _DCH
        idx = [
            pltpu.async_copy(cb0_hbm.at[pl.ds(base, _TPW)], cb0_v, sl0),
            pltpu.async_copy(cb1_hbm.at[pl.ds(base, _TPW)], cb1_v, sl0),
        ]
        for cp in idx:
            cp.wait()

        def _fire(ch, bank):
            r0 = base + ch * _DCH
            sem = sl[bank]
            return [
                pltpu.async_copy(xz_hbm.at[pl.ds(r0, _DCH)], xb[bank], sem),
                pltpu.async_copy(
                    y_hbm.at[cb0_v.at[pl.ds(ch * _DCH, _DCH)]], y0[bank], sem),
                pltpu.async_copy(
                    y_hbm.at[cb1_v.at[pl.ds(ch * _DCH, _DCH)]], y1[bank], sem),
            ]

        loads = {0: _fire(0, 0)}
        stores = {}
        for ch in range(nch):
            bank = ch & 1
            if ch >= 1:
                for cp in stores[ch - 1]:
                    cp.wait()
            if ch + 1 < nch:
                loads[ch + 1] = _fire(ch + 1, bank ^ 1)
            for cp in loads[ch]:
                cp.wait()

            xbb, y0b, y1b = xb[bank], y0[bank], y1[bank]

            def _token(j, _):
                for u in range(_D_MODEL // 16):
                    cols = pl.ds(u * 16, 16)
                    xbb[j, cols] = (xbb[j, cols] + y0b[j, cols]
                                    + y1b[j, cols])
                return 0

            lax.fori_loop(0, _DCH, _token, 0)
            stores[ch] = [
                pltpu.async_copy(xb[bank],
                                 out_hbm.at[pl.ds(base + ch * _DCH, _DCH)],
                                 ss[bank]),
            ]
        for cp in stores[nch - 1]:
            cp.wait()

    return _combine


def kernel(hidden_states, num_global_tokens, max_num_tokens_per_gpu,
           router_weight, correction_bias, w_gate, w_up, w_down):
    x = hidden_states.astype(jnp.float32)
    wrt = jnp.zeros((_D_MODEL, _LANES), jnp.float32).at[:, :_NUM_TOTAL].set(
        router_weight.T.astype(jnp.float32))
    bias_pad = jnp.zeros((1, _LANES), jnp.float32).at[0, :_NUM_TOTAL].set(
        correction_bias.astype(jnp.float32))

    (ssc0, ssc1, scb0, scb1, w0, w1, xz, te, nt) = _route(x, wrt, bias_pad)

    s0 = ssc0.reshape(_N_TOKENS)
    s1 = ssc1.reshape(_N_TOKENS)
    xg, ws = _make_dispatch()(s0, s1, w0, w1, x)

    te_flat = te.reshape(_LANES)[:_GRID_TILES]
    nt_flat = nt.reshape(1)
    y = _grouped_ffn(te_flat, nt_flat, xg, ws,
                     w_gate.astype(jnp.bfloat16),
                     w_up.astype(jnp.bfloat16),
                     w_down.astype(jnp.bfloat16))

    out = _make_combine()(scb0.reshape(_N_TOKENS), scb1.reshape(_N_TOKENS),
                          xz, y)
    return out


# speedup vs baseline: 1.1454x; 1.1454x over previous
import functools

import jax
import jax.numpy as jnp
from jax import lax
from jax.experimental import pallas as pl
from jax.experimental.pallas import tpu as pltpu
from jax.experimental.pallas import tpu_sc as plsc

_NUM_ROUTED = 8
_NUM_TOTAL = 10
_D_MODEL = 1024
_D_FF = 512
_N_TOKENS = 2048
_SCALE = 2.5
_LANES = 128

_TILE = 256
_GRID_TILES = 24
_MAX_PAD = _TILE * _GRID_TILES
_XG_ROWS = _MAX_PAD + 256
_RBLK = 256
_NRB = _N_TOKENS // _RBLK


def _route_body(wrt_ref, bias_ref, x_ref,
                ssc0_ref, ssc1_ref, scb0_ref, scb1_ref,
                w0_ref, w1_ref, xz_ref, te_ref, nt_ref,
                ids_s):
    neg = jnp.float32(-1e30)
    cnt = jnp.zeros((1, _LANES), jnp.float32)

    for tb in range(_NRB):
        rows = pl.ds(tb * _RBLK, _RBLK)
        xb = x_ref[rows, :]
        logits = jnp.dot(xb, wrt_ref[:], preferred_element_type=jnp.float32)
        lane = jax.lax.broadcasted_iota(jnp.int32, logits.shape, 1)
        valid = lane < _NUM_TOTAL
        lm = jnp.where(valid, logits, neg)
        m = jnp.max(lm, axis=-1, keepdims=True)
        p = jnp.where(valid, jnp.exp(lm - m), 0.0)
        scores = p / jnp.sum(p, axis=-1, keepdims=True)
        sel = jnp.where(valid, scores + bias_ref[:], neg)
        m1 = jnp.max(sel, axis=-1, keepdims=True)
        i1 = jnp.min(jnp.where(sel == m1, lane, _LANES), axis=-1, keepdims=True)
        w1v = jnp.sum(jnp.where(lane == i1, scores, 0.0), axis=-1, keepdims=True)
        sel2 = jnp.where(lane == i1, neg, sel)
        m2 = jnp.max(sel2, axis=-1, keepdims=True)
        i2 = jnp.min(jnp.where(sel2 == m2, lane, _LANES), axis=-1, keepdims=True)
        w2v = jnp.sum(jnp.where(lane == i2, scores, 0.0), axis=-1, keepdims=True)
        r1 = i1 < _NUM_ROUTED
        r2 = i2 < _NUM_ROUTED
        ones = jnp.ones((1, _LANES), jnp.float32)
        w0_ref[rows, :] = jnp.where(r1, _SCALE * w1v, 0.0) * ones
        w1_ref[rows, :] = jnp.where(r2, _SCALE * w2v, 0.0) * ones
        zw = _SCALE * (jnp.where(r1, 0.0, w1v) + jnp.where(r2, 0.0, w2v))
        xz_ref[rows, :] = zw * xb
        ids_s[rows, :] = i1
        ids_s[pl.ds(_N_TOKENS + tb * _RBLK, _RBLK), :] = i2
        cnt = cnt + jnp.sum((lane == i1).astype(jnp.float32)
                            + (lane == i2).astype(jnp.float32),
                            axis=0, keepdims=True)

    lane_r = jax.lax.broadcasted_iota(jnp.int32, (1, _LANES), 1)
    pc = jnp.where(lane_r < _NUM_ROUTED,
                   jnp.ceil(cnt / _TILE) * _TILE, 0.0)
    rowi = jax.lax.broadcasted_iota(jnp.int32, (_LANES, _LANES), 0)
    coli = jax.lax.broadcasted_iota(jnp.int32, (_LANES, _LANES), 1)
    upper = (rowi < coli).astype(jnp.float32)
    off = jnp.dot(pc, upper, preferred_element_type=jnp.float32)
    total = off[:, _NUM_ROUTED:_NUM_ROUTED + 1]
    nt_ref[:] = (total / _TILE).astype(jnp.int32)

    te = jnp.zeros((1, _LANES), jnp.float32)
    for e in range(_NUM_ROUTED):
        off_e = off[:, e:e + 1] / _TILE
        te = te + (lane_r.astype(jnp.float32) >= off_e).astype(jnp.float32)
    te_ref[:] = jnp.clip(te - 1.0, 0.0, float(_NUM_ROUTED - 1)).astype(jnp.int32)

    rowb = jax.lax.broadcasted_iota(jnp.int32, (_RBLK, _RBLK), 0)
    colb = jax.lax.broadcasted_iota(jnp.int32, (_RBLK, _RBLK), 1)
    strict_low = (colb < rowb).astype(jnp.float32)
    carry = jnp.zeros((1, _LANES), jnp.float32)
    dump = _MAX_PAD + jax.lax.broadcasted_iota(jnp.int32, (_RBLK, 1), 0)
    for b in range(2 * _NRB):
        ids = ids_s[pl.ds(b * _RBLK, _RBLK), :]
        lane = jax.lax.broadcasted_iota(jnp.int32, (_RBLK, _LANES), 1)
        oh = (lane == ids).astype(jnp.float32)
        prefix = jnp.dot(strict_low, oh, preferred_element_type=jnp.float32)
        grank = jnp.sum(jnp.where(lane == ids, prefix + carry, 0.0),
                        axis=-1, keepdims=True)
        offsel = jnp.sum(jnp.where(lane == ids, off, 0.0),
                         axis=-1, keepdims=True)
        carry = carry + jnp.sum(oh, axis=0, keepdims=True)
        slot = (offsel + grank).astype(jnp.int32)
        routed = ids < _NUM_ROUTED
        ssc = jnp.where(routed, slot, dump)
        scb = jnp.where(routed, slot, dump)
        rows = pl.ds((b % _NRB) * _RBLK, _RBLK)
        if b < _NRB:
            ssc0_ref[rows, :] = ssc
            scb0_ref[rows, :] = scb
        else:
            ssc1_ref[rows, :] = ssc
            scb1_ref[rows, :] = scb


def _route(x, wrt, bias_pad):
    i32 = jnp.int32
    f32 = jnp.float32
    outs = pl.pallas_call(
        _route_body,
        in_specs=[
            pl.BlockSpec((_D_MODEL, _LANES), lambda: (0, 0)),
            pl.BlockSpec((1, _LANES), lambda: (0, 0)),
            pl.BlockSpec((_N_TOKENS, _D_MODEL), lambda: (0, 0)),
        ],
        out_specs=[
            pl.BlockSpec((_N_TOKENS, 1), lambda: (0, 0)),
            pl.BlockSpec((_N_TOKENS, 1), lambda: (0, 0)),
            pl.BlockSpec((_N_TOKENS, 1), lambda: (0, 0)),
            pl.BlockSpec((_N_TOKENS, 1), lambda: (0, 0)),
            pl.BlockSpec((_N_TOKENS, _LANES), lambda: (0, 0)),
            pl.BlockSpec((_N_TOKENS, _LANES), lambda: (0, 0)),
            pl.BlockSpec((_N_TOKENS, _D_MODEL), lambda: (0, 0)),
            pl.BlockSpec((1, _LANES), lambda: (0, 0)),
            pl.BlockSpec((1, 1), lambda: (0, 0)),
        ],
        out_shape=[
            jax.ShapeDtypeStruct((_N_TOKENS, 1), i32),
            jax.ShapeDtypeStruct((_N_TOKENS, 1), i32),
            jax.ShapeDtypeStruct((_N_TOKENS, 1), i32),
            jax.ShapeDtypeStruct((_N_TOKENS, 1), i32),
            jax.ShapeDtypeStruct((_N_TOKENS, _LANES), f32),
            jax.ShapeDtypeStruct((_N_TOKENS, _LANES), f32),
            jax.ShapeDtypeStruct((_N_TOKENS, _D_MODEL), f32),
            jax.ShapeDtypeStruct((1, _LANES), i32),
            jax.ShapeDtypeStruct((1, 1), i32),
        ],
        scratch_shapes=[pltpu.VMEM((2 * _N_TOKENS, 1), i32)],
    )(wrt, bias_pad, x)
    return outs


_NW = 32
_TPW = _N_TOKENS // _NW
_BCH = 32


@functools.cache
def _make_dispatch():
    mesh = plsc.VectorSubcoreMesh(core_axis_name="c", subcore_axis_name="s")

    @functools.partial(
        pl.kernel,
        out_type=[
            jax.ShapeDtypeStruct((_XG_ROWS, _D_MODEL), jnp.float32),
            jax.ShapeDtypeStruct((_XG_ROWS, _LANES), jnp.float32),
        ],
        mesh=mesh,
        scratch_types=[
            pltpu.VMEM((_TPW,), jnp.int32),
            pltpu.VMEM((_TPW,), jnp.int32),
            pltpu.VMEM((_TPW, _LANES), jnp.float32),
            pltpu.VMEM((_TPW, _LANES), jnp.float32),
            pltpu.VMEM((_TPW, _D_MODEL), jnp.float32),
            pltpu.SemaphoreType.DMA,
        ],
    )
    def _dispatch(s0_hbm, s1_hbm, w0_hbm, w1_hbm, x_hbm, xg_hbm, ws_hbm,
                  idx0_v, idx1_v, w0_v, w1_v, rows_v, sem):
        wid = lax.axis_index("s") * 2 + lax.axis_index("c")
        base = wid * _TPW
        loads = [
            pltpu.async_copy(s0_hbm.at[pl.ds(base, _TPW)], idx0_v, sem),
            pltpu.async_copy(s1_hbm.at[pl.ds(base, _TPW)], idx1_v, sem),
            pltpu.async_copy(w0_hbm.at[pl.ds(base, _TPW)], w0_v, sem),
            pltpu.async_copy(w1_hbm.at[pl.ds(base, _TPW)], w1_v, sem),
            pltpu.async_copy(x_hbm.at[pl.ds(base, _TPW)], rows_v, sem),
        ]
        for cp in loads:
            cp.wait()
        stores = [
            pltpu.async_copy(rows_v, xg_hbm.at[idx0_v], sem),
            pltpu.async_copy(rows_v, xg_hbm.at[idx1_v], sem),
            pltpu.async_copy(w0_v, ws_hbm.at[idx0_v], sem),
            pltpu.async_copy(w1_v, ws_hbm.at[idx1_v], sem),
        ]
        for cp in stores:
            cp.wait()

    return _dispatch


def _ffn_body(te_ref, nt_ref, xg_ref, ws_ref, wg_ref, wu_ref, wd_ref, y_ref):
    i = pl.program_id(0)

    @pl.when(i < nt_ref[0])
    def _compute():
        xb = xg_ref[:]
        g = jnp.dot(xb, wg_ref[0], preferred_element_type=jnp.float32)
        u = jnp.dot(xb, wu_ref[0], preferred_element_type=jnp.float32)
        h = g * jax.nn.sigmoid(g) * u
        y = jnp.dot(h, wd_ref[0], preferred_element_type=jnp.float32)
        y_ref[:] = y * ws_ref[:, 0:1]

    @pl.when(i == nt_ref[0])
    def _inactive():
        y_ref[:] = jnp.zeros_like(y_ref)


def _grouped_ffn(te, nt, xg, ws, w_gate, w_up, w_down):
    def _last(i, nt_ref):
        return jnp.minimum(i, jnp.maximum(nt_ref[0] - 1, 0))

    grid_spec = pltpu.PrefetchScalarGridSpec(
        num_scalar_prefetch=2,
        grid=(_GRID_TILES + 1,),
        in_specs=[
            pl.BlockSpec((_TILE, _D_MODEL),
                         lambda i, te_r, nt_r: (_last(i, nt_r), 0)),
            pl.BlockSpec((_TILE, _LANES),
                         lambda i, te_r, nt_r: (_last(i, nt_r), 0)),
            pl.BlockSpec((1, _D_MODEL, _D_FF),
                         lambda i, te_r, nt_r: (te_r[_last(i, nt_r)], 0, 0)),
            pl.BlockSpec((1, _D_MODEL, _D_FF),
                         lambda i, te_r, nt_r: (te_r[_last(i, nt_r)], 0, 0)),
            pl.BlockSpec((1, _D_FF, _D_MODEL),
                         lambda i, te_r, nt_r: (te_r[_last(i, nt_r)], 0, 0)),
        ],
        out_specs=pl.BlockSpec(
            (_TILE, _D_MODEL),
            lambda i, te_r, nt_r: (jnp.where(i < nt_r[0], i, _GRID_TILES), 0)),
    )
    return pl.pallas_call(
        _ffn_body,
        grid_spec=grid_spec,
        out_shape=jax.ShapeDtypeStruct((_XG_ROWS, _D_MODEL), jnp.float32),
    )(te, nt, xg, ws, w_gate, w_up, w_down)


_DCH = 16


@functools.cache
def _make_combine():
    mesh = plsc.VectorSubcoreMesh(core_axis_name="c", subcore_axis_name="s")

    @functools.partial(
        pl.kernel,
        out_type=jax.ShapeDtypeStruct((_N_TOKENS, _D_MODEL), jnp.float32),
        mesh=mesh,
        scratch_types=[
            pltpu.VMEM((_TPW,), jnp.int32),
            pltpu.VMEM((_TPW,), jnp.int32),
            pltpu.VMEM((_DCH, _D_MODEL), jnp.float32),
            pltpu.VMEM((_DCH, _D_MODEL), jnp.float32),
            pltpu.VMEM((_DCH, _D_MODEL), jnp.float32),
            pltpu.VMEM((_DCH, _D_MODEL), jnp.float32),
            pltpu.VMEM((_DCH, _D_MODEL), jnp.float32),
            pltpu.VMEM((_DCH, _D_MODEL), jnp.float32),
            pltpu.SemaphoreType.DMA,
            pltpu.SemaphoreType.DMA,
            pltpu.SemaphoreType.DMA,
            pltpu.SemaphoreType.DMA,
        ],
    )
    def _combine(cb0_hbm, cb1_hbm, xz_hbm, y_hbm, out_hbm,
                 cb0_v, cb1_v, xb0_v, xb1_v, y00_v, y01_v, y10_v, y11_v,
                 sl0, sl1, ss0, ss1):
        wid = lax.axis_index("s") * 2 + lax.axis_index("c")
        base = wid * _TPW
        xb = [xb0_v, xb1_v]
        y0 = [y00_v, y01_v]
        y1 = [y10_v, y11_v]
        sl = [sl0, sl1]
        ss = [ss0, ss1]
        nch = _TPW // _DCH
        idx = [
            pltpu.async_copy(cb0_hbm.at[pl.ds(base, _TPW)], cb0_v, sl0),
            pltpu.async_copy(cb1_hbm.at[pl.ds(base, _TPW)], cb1_v, sl0),
        ]
        for cp in idx:
            cp.wait()

        def _fire(ch, bank):
            r0 = base + ch * _DCH
            sem = sl[bank]
            return [
                pltpu.async_copy(xz_hbm.at[pl.ds(r0, _DCH)], xb[bank], sem),
                pltpu.async_copy(
                    y_hbm.at[cb0_v.at[pl.ds(ch * _DCH, _DCH)]], y0[bank], sem),
                pltpu.async_copy(
                    y_hbm.at[cb1_v.at[pl.ds(ch * _DCH, _DCH)]], y1[bank], sem),
            ]

        loads = {0: _fire(0, 0)}
        stores = {}
        for ch in range(nch):
            bank = ch & 1
            if ch >= 1:
                for cp in stores[ch - 1]:
                    cp.wait()
            if ch + 1 < nch:
                loads[ch + 1] = _fire(ch + 1, bank ^ 1)
            for cp in loads[ch]:
                cp.wait()

            xbb, y0b, y1b = xb[bank], y0[bank], y1[bank]

            def _token(j, _):
                for u in range(_D_MODEL // 16):
                    cols = pl.ds(u * 16, 16)
                    xbb[j, cols] = (xbb[j, cols] + y0b[j, cols]
                                    + y1b[j, cols])
                return 0

            lax.fori_loop(0, _DCH, _token, 0)
            stores[ch] = [
                pltpu.async_copy(xb[bank],
                                 out_hbm.at[pl.ds(base + ch * _DCH, _DCH)],
                                 ss[bank]),
            ]
        for cp in stores[nch - 1]:
            cp.wait()

    return _combine


def kernel(hidden_states, num_global_tokens, max_num_tokens_per_gpu,
           router_weight, correction_bias, w_gate, w_up, w_down):
    x = hidden_states.astype(jnp.float32)
    wrt = jnp.zeros((_D_MODEL, _LANES), jnp.float32).at[:, :_NUM_TOTAL].set(
        router_weight.T.astype(jnp.float32))
    bias_pad = jnp.zeros((1, _LANES), jnp.float32).at[0, :_NUM_TOTAL].set(
        correction_bias.astype(jnp.float32))

    (ssc0, ssc1, scb0, scb1, w0, w1, xz, te, nt) = _route(x, wrt, bias_pad)

    s0 = ssc0.reshape(_N_TOKENS)
    s1 = ssc1.reshape(_N_TOKENS)
    xg, ws = _make_dispatch()(s0, s1, w0, w1, x)

    te_flat = te.reshape(_LANES)[:_GRID_TILES]
    nt_flat = nt.reshape(1)
    y = _grouped_ffn(te_flat, nt_flat, xg, ws, w_gate, w_up, w_down)

    out = _make_combine()(scb0.reshape(_N_TOKENS), scb1.reshape(_N_TOKENS),
                          xz, y)
    return out


# speedup vs baseline: 1.2655x; 1.1049x over previous
import functools

import jax
import jax.numpy as jnp
from jax import lax
from jax.experimental import pallas as pl
from jax.experimental.pallas import tpu as pltpu
from jax.experimental.pallas import tpu_sc as plsc

_NUM_ROUTED = 8
_NUM_TOTAL = 10
_D_MODEL = 1024
_D_FF = 512
_N_TOKENS = 2048
_SCALE = 2.5
_LANES = 128

_TILE = 512
_GRID_TILES = 16
_MAX_PAD = _TILE * _GRID_TILES
_XG_ROWS = _MAX_PAD + _TILE
_RBLK = 256
_NRB = _N_TOKENS // _RBLK


def _route_body(wrt_ref, bias_ref, x_ref,
                ssc0_ref, ssc1_ref, scb0_ref, scb1_ref,
                w0_ref, w1_ref, xz_ref, te_ref, nt_ref,
                ids_s):
    neg = jnp.float32(-1e30)
    cnt = jnp.zeros((1, _LANES), jnp.float32)

    for tb in range(_NRB):
        rows = pl.ds(tb * _RBLK, _RBLK)
        xb = x_ref[rows, :]
        logits = jnp.dot(xb, wrt_ref[:], preferred_element_type=jnp.float32)
        lane = jax.lax.broadcasted_iota(jnp.int32, logits.shape, 1)
        valid = lane < _NUM_TOTAL
        lm = jnp.where(valid, logits, neg)
        m = jnp.max(lm, axis=-1, keepdims=True)
        p = jnp.where(valid, jnp.exp(lm - m), 0.0)
        scores = p / jnp.sum(p, axis=-1, keepdims=True)
        sel = jnp.where(valid, scores + bias_ref[:], neg)
        m1 = jnp.max(sel, axis=-1, keepdims=True)
        i1 = jnp.min(jnp.where(sel == m1, lane, _LANES), axis=-1, keepdims=True)
        w1v = jnp.sum(jnp.where(lane == i1, scores, 0.0), axis=-1, keepdims=True)
        sel2 = jnp.where(lane == i1, neg, sel)
        m2 = jnp.max(sel2, axis=-1, keepdims=True)
        i2 = jnp.min(jnp.where(sel2 == m2, lane, _LANES), axis=-1, keepdims=True)
        w2v = jnp.sum(jnp.where(lane == i2, scores, 0.0), axis=-1, keepdims=True)
        r1 = i1 < _NUM_ROUTED
        r2 = i2 < _NUM_ROUTED
        ones = jnp.ones((1, _LANES), jnp.float32)
        w0_ref[rows, :] = jnp.where(r1, _SCALE * w1v, 0.0) * ones
        w1_ref[rows, :] = jnp.where(r2, _SCALE * w2v, 0.0) * ones
        zw = _SCALE * (jnp.where(r1, 0.0, w1v) + jnp.where(r2, 0.0, w2v))
        xz_ref[rows, :] = zw * xb
        ids_s[rows, :] = i1
        ids_s[pl.ds(_N_TOKENS + tb * _RBLK, _RBLK), :] = i2
        cnt = cnt + jnp.sum((lane == i1).astype(jnp.float32)
                            + (lane == i2).astype(jnp.float32),
                            axis=0, keepdims=True)

    lane_r = jax.lax.broadcasted_iota(jnp.int32, (1, _LANES), 1)
    pc = jnp.where(lane_r < _NUM_ROUTED,
                   jnp.ceil(cnt / _TILE) * _TILE, 0.0)
    rowi = jax.lax.broadcasted_iota(jnp.int32, (_LANES, _LANES), 0)
    coli = jax.lax.broadcasted_iota(jnp.int32, (_LANES, _LANES), 1)
    upper = (rowi < coli).astype(jnp.float32)
    off = jnp.dot(pc, upper, preferred_element_type=jnp.float32)
    total = off[:, _NUM_ROUTED:_NUM_ROUTED + 1]
    nt_ref[:] = (total / _TILE).astype(jnp.int32)

    te = jnp.zeros((1, _LANES), jnp.float32)
    for e in range(_NUM_ROUTED):
        off_e = off[:, e:e + 1] / _TILE
        te = te + (lane_r.astype(jnp.float32) >= off_e).astype(jnp.float32)
    te_ref[:] = jnp.clip(te - 1.0, 0.0, float(_NUM_ROUTED - 1)).astype(jnp.int32)

    rowb = jax.lax.broadcasted_iota(jnp.int32, (_RBLK, _RBLK), 0)
    colb = jax.lax.broadcasted_iota(jnp.int32, (_RBLK, _RBLK), 1)
    strict_low = (colb < rowb).astype(jnp.float32)
    carry = jnp.zeros((1, _LANES), jnp.float32)
    dump = _MAX_PAD + jax.lax.broadcasted_iota(jnp.int32, (_RBLK, 1), 0)
    for b in range(2 * _NRB):
        ids = ids_s[pl.ds(b * _RBLK, _RBLK), :]
        lane = jax.lax.broadcasted_iota(jnp.int32, (_RBLK, _LANES), 1)
        oh = (lane == ids).astype(jnp.float32)
        prefix = jnp.dot(strict_low, oh, preferred_element_type=jnp.float32)
        grank = jnp.sum(jnp.where(lane == ids, prefix + carry, 0.0),
                        axis=-1, keepdims=True)
        offsel = jnp.sum(jnp.where(lane == ids, off, 0.0),
                         axis=-1, keepdims=True)
        carry = carry + jnp.sum(oh, axis=0, keepdims=True)
        slot = (offsel + grank).astype(jnp.int32)
        routed = ids < _NUM_ROUTED
        ssc = jnp.where(routed, slot, dump)
        scb = jnp.where(routed, slot, dump)
        rows = pl.ds((b % _NRB) * _RBLK, _RBLK)
        if b < _NRB:
            ssc0_ref[rows, :] = ssc
            scb0_ref[rows, :] = scb
        else:
            ssc1_ref[rows, :] = ssc
            scb1_ref[rows, :] = scb


def _route(x, wrt, bias_pad):
    i32 = jnp.int32
    f32 = jnp.float32
    outs = pl.pallas_call(
        _route_body,
        in_specs=[
            pl.BlockSpec((_D_MODEL, _LANES), lambda: (0, 0)),
            pl.BlockSpec((1, _LANES), lambda: (0, 0)),
            pl.BlockSpec((_N_TOKENS, _D_MODEL), lambda: (0, 0)),
        ],
        out_specs=[
            pl.BlockSpec((_N_TOKENS, 1), lambda: (0, 0)),
            pl.BlockSpec((_N_TOKENS, 1), lambda: (0, 0)),
            pl.BlockSpec((_N_TOKENS, 1), lambda: (0, 0)),
            pl.BlockSpec((_N_TOKENS, 1), lambda: (0, 0)),
            pl.BlockSpec((_N_TOKENS, _LANES), lambda: (0, 0)),
            pl.BlockSpec((_N_TOKENS, _LANES), lambda: (0, 0)),
            pl.BlockSpec((_N_TOKENS, _D_MODEL), lambda: (0, 0)),
            pl.BlockSpec((1, _LANES), lambda: (0, 0)),
            pl.BlockSpec((1, 1), lambda: (0, 0)),
        ],
        out_shape=[
            jax.ShapeDtypeStruct((_N_TOKENS, 1), i32),
            jax.ShapeDtypeStruct((_N_TOKENS, 1), i32),
            jax.ShapeDtypeStruct((_N_TOKENS, 1), i32),
            jax.ShapeDtypeStruct((_N_TOKENS, 1), i32),
            jax.ShapeDtypeStruct((_N_TOKENS, _LANES), f32),
            jax.ShapeDtypeStruct((_N_TOKENS, _LANES), f32),
            jax.ShapeDtypeStruct((_N_TOKENS, _D_MODEL), f32),
            jax.ShapeDtypeStruct((1, _LANES), i32),
            jax.ShapeDtypeStruct((1, 1), i32),
        ],
        scratch_shapes=[pltpu.VMEM((2 * _N_TOKENS, 1), i32)],
    )(wrt, bias_pad, x)
    return outs


_NW = 32
_TPW = _N_TOKENS // _NW
_BCH = 32


@functools.cache
def _make_dispatch():
    mesh = plsc.VectorSubcoreMesh(core_axis_name="c", subcore_axis_name="s")

    @functools.partial(
        pl.kernel,
        out_type=[
            jax.ShapeDtypeStruct((_XG_ROWS, _D_MODEL), jnp.float32),
            jax.ShapeDtypeStruct((_XG_ROWS, _LANES), jnp.float32),
        ],
        mesh=mesh,
        scratch_types=[
            pltpu.VMEM((_TPW,), jnp.int32),
            pltpu.VMEM((_TPW,), jnp.int32),
            pltpu.VMEM((_TPW, _LANES), jnp.float32),
            pltpu.VMEM((_TPW, _LANES), jnp.float32),
            pltpu.VMEM((_TPW, _D_MODEL), jnp.float32),
            pltpu.SemaphoreType.DMA,
        ],
    )
    def _dispatch(s0_hbm, s1_hbm, w0_hbm, w1_hbm, x_hbm, xg_hbm, ws_hbm,
                  idx0_v, idx1_v, w0_v, w1_v, rows_v, sem):
        wid = lax.axis_index("s") * 2 + lax.axis_index("c")
        base = wid * _TPW
        loads = [
            pltpu.async_copy(s0_hbm.at[pl.ds(base, _TPW)], idx0_v, sem),
            pltpu.async_copy(s1_hbm.at[pl.ds(base, _TPW)], idx1_v, sem),
            pltpu.async_copy(w0_hbm.at[pl.ds(base, _TPW)], w0_v, sem),
            pltpu.async_copy(w1_hbm.at[pl.ds(base, _TPW)], w1_v, sem),
            pltpu.async_copy(x_hbm.at[pl.ds(base, _TPW)], rows_v, sem),
        ]
        for cp in loads:
            cp.wait()
        stores = [
            pltpu.async_copy(rows_v, xg_hbm.at[idx0_v], sem),
            pltpu.async_copy(rows_v, xg_hbm.at[idx1_v], sem),
            pltpu.async_copy(w0_v, ws_hbm.at[idx0_v], sem),
            pltpu.async_copy(w1_v, ws_hbm.at[idx1_v], sem),
        ]
        for cp in stores:
            cp.wait()

    return _dispatch


def _ffn_body(te_ref, nt_ref, xg_ref, ws_ref, wg_ref, wu_ref, wd_ref, y_ref):
    i = pl.program_id(0)

    @pl.when(i < nt_ref[0])
    def _compute():
        xb = xg_ref[:]
        g = jnp.dot(xb, wg_ref[0], preferred_element_type=jnp.float32)
        u = jnp.dot(xb, wu_ref[0], preferred_element_type=jnp.float32)
        h = g * jax.nn.sigmoid(g) * u
        y = jnp.dot(h, wd_ref[0], preferred_element_type=jnp.float32)
        y_ref[:] = y * ws_ref[:, 0:1]

    @pl.when(i == nt_ref[0])
    def _inactive():
        y_ref[:] = jnp.zeros_like(y_ref)


def _grouped_ffn(te, nt, xg, ws, w_gate, w_up, w_down):
    def _last(i, nt_ref):
        return jnp.minimum(i, jnp.maximum(nt_ref[0] - 1, 0))

    grid_spec = pltpu.PrefetchScalarGridSpec(
        num_scalar_prefetch=2,
        grid=(_GRID_TILES + 1,),
        in_specs=[
            pl.BlockSpec((_TILE, _D_MODEL),
                         lambda i, te_r, nt_r: (_last(i, nt_r), 0)),
            pl.BlockSpec((_TILE, _LANES),
                         lambda i, te_r, nt_r: (_last(i, nt_r), 0)),
            pl.BlockSpec((1, _D_MODEL, _D_FF),
                         lambda i, te_r, nt_r: (te_r[_last(i, nt_r)], 0, 0)),
            pl.BlockSpec((1, _D_MODEL, _D_FF),
                         lambda i, te_r, nt_r: (te_r[_last(i, nt_r)], 0, 0)),
            pl.BlockSpec((1, _D_FF, _D_MODEL),
                         lambda i, te_r, nt_r: (te_r[_last(i, nt_r)], 0, 0)),
        ],
        out_specs=pl.BlockSpec(
            (_TILE, _D_MODEL),
            lambda i, te_r, nt_r: (jnp.where(i < nt_r[0], i, _GRID_TILES), 0)),
    )
    return pl.pallas_call(
        _ffn_body,
        grid_spec=grid_spec,
        out_shape=jax.ShapeDtypeStruct((_XG_ROWS, _D_MODEL), jnp.float32),
    )(te, nt, xg, ws, w_gate, w_up, w_down)


_DCH = 16


@functools.cache
def _make_combine():
    mesh = plsc.VectorSubcoreMesh(core_axis_name="c", subcore_axis_name="s")

    @functools.partial(
        pl.kernel,
        out_type=jax.ShapeDtypeStruct((_N_TOKENS, _D_MODEL), jnp.float32),
        mesh=mesh,
        scratch_types=[
            pltpu.VMEM((_TPW,), jnp.int32),
            pltpu.VMEM((_TPW,), jnp.int32),
            pltpu.VMEM((_DCH, _D_MODEL), jnp.float32),
            pltpu.VMEM((_DCH, _D_MODEL), jnp.float32),
            pltpu.VMEM((_DCH, _D_MODEL), jnp.float32),
            pltpu.VMEM((_DCH, _D_MODEL), jnp.float32),
            pltpu.VMEM((_DCH, _D_MODEL), jnp.float32),
            pltpu.VMEM((_DCH, _D_MODEL), jnp.float32),
            pltpu.SemaphoreType.DMA,
            pltpu.SemaphoreType.DMA,
            pltpu.SemaphoreType.DMA,
            pltpu.SemaphoreType.DMA,
        ],
    )
    def _combine(cb0_hbm, cb1_hbm, xz_hbm, y_hbm, out_hbm,
                 cb0_v, cb1_v, xb0_v, xb1_v, y00_v, y01_v, y10_v, y11_v,
                 sl0, sl1, ss0, ss1):
        wid = lax.axis_index("s") * 2 + lax.axis_index("c")
        base = wid * _TPW
        xb = [xb0_v, xb1_v]
        y0 = [y00_v, y01_v]
        y1 = [y10_v, y11_v]
        sl = [sl0, sl1]
        ss = [ss0, ss1]
        nch = _TPW // _DCH
        idx = [
            pltpu.async_copy(cb0_hbm.at[pl.ds(base, _TPW)], cb0_v, sl0),
            pltpu.async_copy(cb1_hbm.at[pl.ds(base, _TPW)], cb1_v, sl0),
        ]
        for cp in idx:
            cp.wait()

        def _fire(ch, bank):
            r0 = base + ch * _DCH
            sem = sl[bank]
            return [
                pltpu.async_copy(xz_hbm.at[pl.ds(r0, _DCH)], xb[bank], sem),
                pltpu.async_copy(
                    y_hbm.at[cb0_v.at[pl.ds(ch * _DCH, _DCH)]], y0[bank], sem),
                pltpu.async_copy(
                    y_hbm.at[cb1_v.at[pl.ds(ch * _DCH, _DCH)]], y1[bank], sem),
            ]

        loads = {0: _fire(0, 0)}
        stores = {}
        for ch in range(nch):
            bank = ch & 1
            if ch >= 1:
                for cp in stores[ch - 1]:
                    cp.wait()
            if ch + 1 < nch:
                loads[ch + 1] = _fire(ch + 1, bank ^ 1)
            for cp in loads[ch]:
                cp.wait()

            xbb, y0b, y1b = xb[bank], y0[bank], y1[bank]

            def _token(j, _):
                for u in range(_D_MODEL // 16):
                    cols = pl.ds(u * 16, 16)
                    xbb[j, cols] = (xbb[j, cols] + y0b[j, cols]
                                    + y1b[j, cols])
                return 0

            lax.fori_loop(0, _DCH, _token, 0)
            stores[ch] = [
                pltpu.async_copy(xb[bank],
                                 out_hbm.at[pl.ds(base + ch * _DCH, _DCH)],
                                 ss[bank]),
            ]
        for cp in stores[nch - 1]:
            cp.wait()

    return _combine


def kernel(hidden_states, num_global_tokens, max_num_tokens_per_gpu,
           router_weight, correction_bias, w_gate, w_up, w_down):
    x = hidden_states.astype(jnp.float32)
    wrt = jnp.zeros((_D_MODEL, _LANES), jnp.float32).at[:, :_NUM_TOTAL].set(
        router_weight.T.astype(jnp.float32))
    bias_pad = jnp.zeros((1, _LANES), jnp.float32).at[0, :_NUM_TOTAL].set(
        correction_bias.astype(jnp.float32))

    (ssc0, ssc1, scb0, scb1, w0, w1, xz, te, nt) = _route(x, wrt, bias_pad)

    s0 = ssc0.reshape(_N_TOKENS)
    s1 = ssc1.reshape(_N_TOKENS)
    xg, ws = _make_dispatch()(s0, s1, w0, w1, x)

    te_flat = te.reshape(_LANES)[:_GRID_TILES]
    nt_flat = nt.reshape(1)
    y = _grouped_ffn(te_flat, nt_flat, xg, ws, w_gate, w_up, w_down)

    out = _make_combine()(scb0.reshape(_N_TOKENS), scb1.reshape(_N_TOKENS),
                          xz, y)
    return out
